# baseline (speedup 1.0000x reference)
import functools

import jax
import jax.numpy as jnp
from jax import lax
from jax.experimental import pallas as pl
from jax.experimental.pallas import tpu as pltpu
from jax.experimental.pallas import tpu_sc as plsc

L = 16
NEG_INF = float("-inf")


def _scalar_at(ref, idx):
    return ref[pl.ds(idx, L)][0]


@functools.lru_cache(maxsize=None)
def _build(batch: int, vocab: int):
    info = plsc.get_sparse_core_info()
    nc, ns = info.num_cores, info.num_subcores
    nw = nc * ns
    assert nc == 2 and ns == 16
    assert batch == 2 * nw, (batch, nw)
    rows_per_w = batch // nw
    assert vocab % L == 0 and vocab % 8 == 0
    nvreg = vocab // L

    WIN = 4096
    SL = WIN // 8
    acols = (vocab // 128) * 128
    tailw = vocab - acols
    nfull = acols // WIN
    rem = acols - nfull * WIN
    assert nfull >= 4 and nfull % 2 == 0 and rem > 0
    rem_t = [min(max(rem - 256 * t, 0), 256) for t in range(8)]
    assert sum(rem_t) == rem and all(w % 128 == 0 for w in rem_t)
    assert tailw % 8 == 0 and (rem + tailw) % 8 == 0

    mesh = plsc.VectorSubcoreMesh(core_axis_name="c", subcore_axis_name="s")

    @functools.partial(
        pl.kernel,
        out_type=(jax.ShapeDtypeStruct((batch, vocab), jnp.float32),
                  jax.ShapeDtypeStruct((batch * tailw,), jnp.float32)),
        mesh=mesh,
        compiler_params=pltpu.CompilerParams(needs_layout_passes=False),
        scratch_types=[
            pltpu.VMEM((vocab + L,), jnp.float32),
            pltpu.VMEM((WIN,), jnp.int32),
            pltpu.VMEM((WIN,), jnp.float32),
            pltpu.VMEM((WIN,), jnp.float32),
            pltpu.VMEM((batch + L,), jnp.float32),
            pltpu.VMEM((batch + L,), jnp.int32),
            pltpu.VMEM_SHARED((2, 2, 8, WIN), jnp.int32),
            pltpu.VMEM_SHARED((2, 2, 8, WIN), jnp.float32),
            pltpu.SemaphoreType.DMA,
            pltpu.SemaphoreType.DMA,
            pltpu.SemaphoreType.DMA,
        ],
    )
    def sc_kernel(sv_hbm, svt_hbm, si_hbm, sit_hbm, p_hbm, k_hbm,
                  out_hbm, ot_hbm,
                  row_v, idx_v, out_v0, out_v1, p_v, k_v,
                  shr_si, shr_out, sem_st, sem_sv, sem_fl):
        sid = lax.axis_index("s")
        cid = lax.axis_index("c")
        grp = sid // 8
        s8 = sid % 8
        pltpu.sync_copy(p_hbm, p_v.at[pl.ds(0, batch)])
        pltpu.sync_copy(k_hbm, k_v.at[pl.ds(0, batch)])

        out_v = (out_v0, out_v1)

        for r in range(rows_per_w):
            blk = cid * 4 + r * 2 + grp
            row = blk * 8 + s8
            base = row * vocab

            def stage_full(w, b):
                col = pl.multiple_of(w * WIN + s8 * SL, 128)
                pltpu.async_copy(
                    si_hbm.at[pl.ds(blk * 8, 8), pl.ds(col, SL)],
                    shr_si.at[b, grp, :, pl.ds(s8 * SL, SL)], sem_st)

            def wait_full():
                pltpu.make_async_copy(
                    si_hbm.at[pl.ds(blk * 8, 8), pl.ds(0, SL)],
                    shr_si.at[0, grp, :, pl.ds(0, SL)], sem_st).wait()

            def stage_rem():
                for tt, wt in enumerate(rem_t):
                    if wt == 0:
                        continue

                    @pl.when(s8 == tt)
                    def _():
                        pltpu.async_copy(
                            si_hbm.at[pl.ds(blk * 8, 8),
                                      pl.ds(nfull * WIN + 256 * tt, wt)],
                            shr_si.at[0, grp, :, pl.ds(256 * tt, wt)], sem_st)

            def wait_rem():
                for tt, wt in enumerate(rem_t):
                    if wt == 0:
                        continue

                    @pl.when(s8 == tt)
                    def _():
                        pltpu.make_async_copy(
                            si_hbm.at[pl.ds(blk * 8, 8), pl.ds(0, wt)],
                            shr_si.at[0, grp, :, pl.ds(0, wt)],
                            sem_st).wait()

            def fill(b, n):
                pltpu.sync_copy(out_v[b].at[pl.ds(0, n)],
                                shr_out.at[b, grp, s8, pl.ds(0, n)])

            def flush(w, b):
                col = pl.multiple_of(w * WIN + s8 * SL, 128)
                pltpu.async_copy(
                    shr_out.at[b, grp, :, pl.ds(s8 * SL, SL)],
                    out_hbm.at[pl.ds(blk * 8, 8), pl.ds(col, SL)], sem_fl)

            def wait_flush():
                pltpu.make_async_copy(
                    shr_out.at[0, grp, :, pl.ds(0, SL)],
                    out_hbm.at[pl.ds(blk * 8, 8), pl.ds(0, SL)],
                    sem_fl).wait()

            def flush_rem():
                for tt, wt in enumerate(rem_t):
                    if wt == 0:
                        continue

                    @pl.when(s8 == tt)
                    def _():
                        pltpu.async_copy(
                            shr_out.at[0, grp, :, pl.ds(256 * tt, wt)],
                            out_hbm.at[pl.ds(blk * 8, 8),
                                       pl.ds(nfull * WIN + 256 * tt, wt)],
                            sem_fl)

            def wait_flush_rem():
                for tt, wt in enumerate(rem_t):
                    if wt == 0:
                        continue

                    @pl.when(s8 == tt)
                    def _():
                        pltpu.make_async_copy(
                            shr_out.at[0, grp, :, pl.ds(0, wt)],
                            out_hbm.at[pl.ds(blk * 8, 8), pl.ds(0, wt)],
                            sem_fl).wait()

            def stage_sv(w, b):
                col = pl.multiple_of(w * WIN + s8 * SL, 128)
                pltpu.async_copy(
                    sv_hbm.at[pl.ds(blk * 8, 8), pl.ds(col, SL)],
                    shr_out.at[b, grp, :, pl.ds(s8 * SL, SL)], sem_sv)

            def wait_sv():
                pltpu.make_async_copy(
                    sv_hbm.at[pl.ds(blk * 8, 8), pl.ds(0, SL)],
                    shr_out.at[0, grp, :, pl.ds(0, SL)], sem_sv).wait()

            def stage_sv_rem():
                for tt, wt in enumerate(rem_t):
                    if wt == 0:
                        continue

                    @pl.when(s8 == tt)
                    def _():
                        pltpu.async_copy(
                            sv_hbm.at[pl.ds(blk * 8, 8),
                                      pl.ds(nfull * WIN + 256 * tt, wt)],
                            shr_out.at[0, grp, :, pl.ds(256 * tt, wt)],
                            sem_sv)

            def wait_sv_rem():
                for tt, wt in enumerate(rem_t):
                    if wt == 0:
                        continue

                    @pl.when(s8 == tt)
                    def _():
                        pltpu.make_async_copy(
                            sv_hbm.at[pl.ds(blk * 8, 8), pl.ds(0, wt)],
                            shr_out.at[0, grp, :, pl.ds(0, wt)],
                            sem_sv).wait()

            def extract_sv(w, b, n):
                pltpu.sync_copy(shr_out.at[b, grp, s8, pl.ds(0, n)],
                                row_v.at[pl.ds(w * WIN, n)])

            stage_full(0, 0)
            stage_full(1, 1)

            stage_sv(0, 0)
            stage_sv(1, 1)
            wait_sv()
            plsc.subcore_barrier()

            def sv_pair(q, _):
                w0 = 2 * q

                @pl.when(q > 0)
                def _():
                    stage_sv(w0 + 1, 1)

                extract_sv(w0, 0, WIN)
                wait_sv()
                plsc.subcore_barrier()

                stage_sv(w0 + 2, 0)
                extract_sv(w0 + 1, 1, WIN)
                wait_sv()
                plsc.subcore_barrier()
                return 0

            lax.fori_loop(0, (nfull - 2) // 2, sv_pair, 0)

            stage_sv(nfull - 1, 1)
            extract_sv(nfull - 2, 0, WIN)
            wait_sv()
            plsc.subcore_barrier()

            stage_sv_rem()
            extract_sv(nfull - 1, 1, WIN)
            wait_sv_rem()
            plsc.subcore_barrier()

            extract_sv(nfull, 0, rem)
            pltpu.sync_copy(svt_hbm.at[pl.ds(row * tailw, tailw)],
                            row_v.at[pl.ds(acols, tailw)])
            plsc.subcore_barrier()

            kk = _scalar_at(k_v, row)
            pp = _scalar_at(p_v, row)
            m = row_v[pl.ds(vocab - L, L)][L - 1]

            valid = kk >= 1
            idx_t = jnp.clip(vocab - kk, 0, vocab - 1)
            thresh = _scalar_at(row_v, idx_t)
            lo = jnp.int32(0)
            hi = jnp.int32(vocab)
            for _ in range(17):
                cont = lo < hi
                mid = (lo + hi) // 2
                vm = _scalar_at(row_v, jnp.minimum(mid, vocab - 1))
                below = vm < thresh
                lo = jnp.where(cont & below, mid + 1, lo)
                hi = jnp.where(cont & (~below), mid, hi)
            start = jnp.where(valid, lo, 0)
            g0 = start // L

            def sweep_a(g, acc):
                vv = row_v[pl.ds(g * L, L)]
                jj = lax.iota(jnp.int32, L) + g * L
                e = jnp.where(jj >= start, jnp.exp(vv - m), 0.0)
                return acc + e

            acc = lax.fori_loop(g0, nvreg, sweep_a,
                                jnp.zeros((L,), jnp.float32))
            total = jnp.sum(acc)
            t = (1.0 - pp) * total

            def sweep_b(g, carry):
                s, cnt = carry
                vv = row_v[pl.ds(g * L, L)]
                jj = lax.iota(jnp.int32, L) + g * L
                e = jnp.where(jj >= start, jnp.exp(vv - m), 0.0)
                pc = plsc.cumsum(e) + s
                cond = (pc <= t) & (jj >= start)
                cnt = cnt + jnp.sum(cond.astype(jnp.int32))
                return s + jnp.sum(e), cnt

            _, cnt = lax.fori_loop(g0, nvreg, sweep_b,
                                   (jnp.float32(0.0), jnp.int32(0)))
            c = start + cnt

            def extract(b, n):
                pltpu.sync_copy(shr_si.at[b, grp, s8, pl.ds(0, n)],
                                idx_v.at[pl.ds(0, n)])

            def gather(b, n):
                @plsc.parallel_loop(0, n, step=L, unroll=8)
                def gbody(i):
                    idx16 = idx_v[pl.ds(i, L)]
                    vals = plsc.load_gather(row_v, [idx16])
                    keep = (idx16 >= c) | (idx16 == vocab - 1)
                    out_v[b][pl.ds(i, L)] = jnp.where(keep, vals, NEG_INF)

            wait_full()
            plsc.subcore_barrier()

            def pair(q, _):
                w0 = 2 * q
                @pl.when(q > 0)
                def _():
                    stage_full(w0 + 1, 1)
                    flush(w0 - 1, 1)

                extract(0, WIN)
                gather(0, WIN)
                fill(0, WIN)
                wait_full()

                @pl.when(q > 0)
                def _():
                    wait_flush()

                plsc.subcore_barrier()

                stage_full(w0 + 2, 0)
                flush(w0, 0)
                extract(1, WIN)
                gather(1, WIN)
                fill(1, WIN)
                wait_full()
                wait_flush()
                plsc.subcore_barrier()
                return 0

            lax.fori_loop(0, (nfull - 2) // 2, pair, 0)

            stage_full(nfull - 1, 1)
            flush(nfull - 3, 1)
            extract(0, WIN)
            gather(0, WIN)
            fill(0, WIN)
            wait_full()
            wait_flush()
            plsc.subcore_barrier()

            stage_rem()
            flush(nfull - 2, 0)
            extract(1, WIN)
            gather(1, WIN)
            fill(1, WIN)
            wait_rem()
            wait_flush()
            plsc.subcore_barrier()

            flush(nfull - 1, 1)
            extract(0, rem)
            pltpu.sync_copy(sit_hbm.at[pl.ds(row * tailw, tailw)],
                            idx_v.at[pl.ds(rem, tailw)])
            gather(0, rem + tailw)
            fill(0, rem)
            pltpu.sync_copy(out_v[0].at[pl.ds(rem, tailw)],
                            ot_hbm.at[pl.ds(row * tailw, tailw)])
            wait_flush()
            plsc.subcore_barrier()

            flush_rem()
            wait_flush_rem()
            plsc.subcore_barrier()


    return sc_kernel


def kernel(sorted_value, sorted_indices, p, k):
    batch, vocab = sorted_value.shape
    fn = _build(batch, vocab)
    acols = (vocab // 128) * 128
    tailw = vocab - acols
    si = sorted_indices.astype(jnp.int32)
    out2d, otail = fn(sorted_value, sorted_value[:, acols:].reshape(-1),
                      si, si[:, acols:].reshape(-1),
                      p.astype(jnp.float32), k.astype(jnp.int32))
    return lax.dynamic_update_slice(out2d, otail.reshape(batch, tailw),
                                    (0, acols))


# speedup vs baseline: 1.0898x; 1.0898x over previous
import functools

import jax
import jax.numpy as jnp
from jax import lax
from jax.experimental import pallas as pl
from jax.experimental.pallas import tpu as pltpu
from jax.experimental.pallas import tpu_sc as plsc

L = 16
NEG_INF = float("-inf")


def _scalar_at(ref, idx):
    return ref[pl.ds(idx, L)][0]


@functools.lru_cache(maxsize=None)
def _build(batch: int, vocab: int):
    info = plsc.get_sparse_core_info()
    nc, ns = info.num_cores, info.num_subcores
    nw = nc * ns
    assert nc == 2 and ns == 16
    assert batch == 2 * nw, (batch, nw)
    rows_per_w = batch // nw
    assert vocab % L == 0 and vocab % 8 == 0
    nvreg = vocab // L

    WIN = 4096
    SL = WIN // 8
    acols = (vocab // 128) * 128
    tailw = vocab - acols
    nfull = acols // WIN
    rem = acols - nfull * WIN
    assert nfull >= 4 and nfull % 2 == 0 and rem > 0
    rem_t = [min(max(rem - 256 * t, 0), 256) for t in range(8)]
    assert sum(rem_t) == rem and all(w % 128 == 0 for w in rem_t)
    assert tailw % 8 == 0 and (rem + tailw) % 8 == 0

    mesh = plsc.VectorSubcoreMesh(core_axis_name="c", subcore_axis_name="s")

    @functools.partial(
        pl.kernel,
        out_type=(jax.ShapeDtypeStruct((batch, vocab), jnp.float32),
                  jax.ShapeDtypeStruct((batch * tailw,), jnp.float32)),
        mesh=mesh,
        compiler_params=pltpu.CompilerParams(needs_layout_passes=False),
        scratch_types=[
            pltpu.VMEM((vocab + L,), jnp.float32),
            pltpu.VMEM((WIN,), jnp.int32),
            pltpu.VMEM((WIN,), jnp.float32),
            pltpu.VMEM((WIN,), jnp.float32),
            pltpu.VMEM((batch + L,), jnp.float32),
            pltpu.VMEM((batch + L,), jnp.int32),
            pltpu.VMEM_SHARED((2, 2, 8, WIN), jnp.int32),
            pltpu.VMEM_SHARED((2, 2, 8, WIN), jnp.float32),
            pltpu.SemaphoreType.DMA,
            pltpu.SemaphoreType.DMA,
        ],
    )
    def sc_kernel(sv_hbm, si_hbm, sit_hbm, p_hbm, k_hbm, out_hbm, ot_hbm,
                  row_v, idx_v, out_v0, out_v1, p_v, k_v,
                  shr_si, shr_out, sem_st, sem_fl):
        sid = lax.axis_index("s")
        cid = lax.axis_index("c")
        grp = sid // 8
        s8 = sid % 8
        pltpu.sync_copy(p_hbm, p_v.at[pl.ds(0, batch)])
        pltpu.sync_copy(k_hbm, k_v.at[pl.ds(0, batch)])

        out_v = (out_v0, out_v1)

        for r in range(rows_per_w):
            blk = cid * 4 + r * 2 + grp
            row = blk * 8 + s8
            base = row * vocab

            def stage_full(w, b):
                col = pl.multiple_of(w * WIN + s8 * SL, 128)
                pltpu.async_copy(
                    si_hbm.at[pl.ds(blk * 8, 8), pl.ds(col, SL)],
                    shr_si.at[b, grp, :, pl.ds(s8 * SL, SL)], sem_st)

            def wait_full():
                pltpu.make_async_copy(
                    si_hbm.at[pl.ds(blk * 8, 8), pl.ds(0, SL)],
                    shr_si.at[0, grp, :, pl.ds(0, SL)], sem_st).wait()

            def stage_rem():
                for tt, wt in enumerate(rem_t):
                    if wt == 0:
                        continue

                    @pl.when(s8 == tt)
                    def _():
                        pltpu.async_copy(
                            si_hbm.at[pl.ds(blk * 8, 8),
                                      pl.ds(nfull * WIN + 256 * tt, wt)],
                            shr_si.at[0, grp, :, pl.ds(256 * tt, wt)], sem_st)

            def wait_rem():
                for tt, wt in enumerate(rem_t):
                    if wt == 0:
                        continue

                    @pl.when(s8 == tt)
                    def _():
                        pltpu.make_async_copy(
                            si_hbm.at[pl.ds(blk * 8, 8), pl.ds(0, wt)],
                            shr_si.at[0, grp, :, pl.ds(0, wt)],
                            sem_st).wait()

            def fill(b, n):
                pltpu.sync_copy(out_v[b].at[pl.ds(0, n)],
                                shr_out.at[b, grp, s8, pl.ds(0, n)])

            def flush(w, b):
                col = pl.multiple_of(w * WIN + s8 * SL, 128)
                pltpu.async_copy(
                    shr_out.at[b, grp, :, pl.ds(s8 * SL, SL)],
                    out_hbm.at[pl.ds(blk * 8, 8), pl.ds(col, SL)], sem_fl)

            def wait_flush():
                pltpu.make_async_copy(
                    shr_out.at[0, grp, :, pl.ds(0, SL)],
                    out_hbm.at[pl.ds(blk * 8, 8), pl.ds(0, SL)],
                    sem_fl).wait()

            def flush_rem():
                for tt, wt in enumerate(rem_t):
                    if wt == 0:
                        continue

                    @pl.when(s8 == tt)
                    def _():
                        pltpu.async_copy(
                            shr_out.at[0, grp, :, pl.ds(256 * tt, wt)],
                            out_hbm.at[pl.ds(blk * 8, 8),
                                       pl.ds(nfull * WIN + 256 * tt, wt)],
                            sem_fl)

            def wait_flush_rem():
                for tt, wt in enumerate(rem_t):
                    if wt == 0:
                        continue

                    @pl.when(s8 == tt)
                    def _():
                        pltpu.make_async_copy(
                            shr_out.at[0, grp, :, pl.ds(0, wt)],
                            out_hbm.at[pl.ds(blk * 8, 8), pl.ds(0, wt)],
                            sem_fl).wait()

            stage_full(0, 0)
            stage_full(1, 1)
            pltpu.sync_copy(sv_hbm.at[pl.ds(base, vocab)],
                            row_v.at[pl.ds(0, vocab)])

            kk = _scalar_at(k_v, row)
            pp = _scalar_at(p_v, row)
            m = row_v[pl.ds(vocab - L, L)][L - 1]

            valid = kk >= 1
            idx_t = jnp.clip(vocab - kk, 0, vocab - 1)
            thresh = _scalar_at(row_v, idx_t)
            lo = jnp.int32(0)
            hi = jnp.int32(vocab)
            for _ in range(17):
                cont = lo < hi
                mid = (lo + hi) // 2
                vm = _scalar_at(row_v, jnp.minimum(mid, vocab - 1))
                below = vm < thresh
                lo = jnp.where(cont & below, mid + 1, lo)
                hi = jnp.where(cont & (~below), mid, hi)
            start = jnp.where(valid, lo, 0)
            g0 = start // L

            def sweep_a(g, acc):
                vv = row_v[pl.ds(g * L, L)]
                jj = lax.iota(jnp.int32, L) + g * L
                e = jnp.where(jj >= start, jnp.exp(vv - m), 0.0)
                return acc + e

            acc = lax.fori_loop(g0, nvreg, sweep_a,
                                jnp.zeros((L,), jnp.float32))
            total = jnp.sum(acc)
            t = (1.0 - pp) * total

            def sweep_b(g, carry):
                s, cnt = carry
                vv = row_v[pl.ds(g * L, L)]
                jj = lax.iota(jnp.int32, L) + g * L
                e = jnp.where(jj >= start, jnp.exp(vv - m), 0.0)
                pc = plsc.cumsum(e) + s
                cond = (pc <= t) & (jj >= start)
                cnt = cnt + jnp.sum(cond.astype(jnp.int32))
                return s + jnp.sum(e), cnt

            _, cnt = lax.fori_loop(g0, nvreg, sweep_b,
                                   (jnp.float32(0.0), jnp.int32(0)))
            c = start + cnt

            def extract(b, n):
                pltpu.sync_copy(shr_si.at[b, grp, s8, pl.ds(0, n)],
                                idx_v.at[pl.ds(0, n)])

            def gather(b, n):
                @plsc.parallel_loop(0, n, step=L, unroll=16)
                def gbody(i):
                    idx16 = idx_v[pl.ds(i, L)]
                    vals = plsc.load_gather(row_v, [idx16])
                    keep = (idx16 >= c) | (idx16 == vocab - 1)
                    out_v[b][pl.ds(i, L)] = jnp.where(keep, vals, NEG_INF)

            wait_full()
            plsc.subcore_barrier()

            def pair(q, _):
                w0 = 2 * q
                @pl.when(q > 0)
                def _():
                    stage_full(w0 + 1, 1)
                    flush(w0 - 1, 1)

                extract(0, WIN)
                gather(0, WIN)
                fill(0, WIN)
                wait_full()

                @pl.when(q > 0)
                def _():
                    wait_flush()

                plsc.subcore_barrier()

                stage_full(w0 + 2, 0)
                flush(w0, 0)
                extract(1, WIN)
                gather(1, WIN)
                fill(1, WIN)
                wait_full()
                wait_flush()
                plsc.subcore_barrier()
                return 0

            lax.fori_loop(0, (nfull - 2) // 2, pair, 0)

            stage_full(nfull - 1, 1)
            flush(nfull - 3, 1)
            extract(0, WIN)
            gather(0, WIN)
            fill(0, WIN)
            wait_full()
            wait_flush()
            plsc.subcore_barrier()

            stage_rem()
            flush(nfull - 2, 0)
            extract(1, WIN)
            gather(1, WIN)
            fill(1, WIN)
            wait_rem()
            wait_flush()
            plsc.subcore_barrier()

            flush(nfull - 1, 1)
            extract(0, rem)
            pltpu.sync_copy(sit_hbm.at[pl.ds(row * tailw, tailw)],
                            idx_v.at[pl.ds(rem, tailw)])
            gather(0, rem + tailw)
            fill(0, rem)
            pltpu.sync_copy(out_v[0].at[pl.ds(rem, tailw)],
                            ot_hbm.at[pl.ds(row * tailw, tailw)])
            wait_flush()
            plsc.subcore_barrier()

            flush_rem()
            wait_flush_rem()


    return sc_kernel


def kernel(sorted_value, sorted_indices, p, k):
    batch, vocab = sorted_value.shape
    fn = _build(batch, vocab)
    acols = (vocab // 128) * 128
    tailw = vocab - acols
    si = sorted_indices.astype(jnp.int32)
    out2d, otail = fn(sorted_value.reshape(-1), si, si[:, acols:].reshape(-1),
                      p.astype(jnp.float32), k.astype(jnp.int32))
    return lax.dynamic_update_slice(out2d, otail.reshape(batch, tailw),
                                    (0, acols))


# speedup vs baseline: 1.0997x; 1.0091x over previous
import functools

import jax
import jax.numpy as jnp
from jax import lax
from jax.experimental import pallas as pl
from jax.experimental.pallas import tpu as pltpu
from jax.experimental.pallas import tpu_sc as plsc

L = 16
NEG_INF = float("-inf")


def _scalar_at(ref, idx):
    return ref[pl.ds(idx, L)][0]


@functools.lru_cache(maxsize=None)
def _build(batch: int, vocab: int):
    info = plsc.get_sparse_core_info()
    nc, ns = info.num_cores, info.num_subcores
    nw = nc * ns
    assert nc == 2 and ns == 16
    assert batch == 2 * nw, (batch, nw)
    rows_per_w = batch // nw
    assert vocab % L == 0 and vocab % 8 == 0
    nvreg = vocab // L

    WIN = 4096
    SL = WIN // 8
    acols = (vocab // 128) * 128
    tailw = vocab - acols
    nfull = acols // WIN
    rem = acols - nfull * WIN
    assert nfull >= 4 and nfull % 2 == 0 and rem > 0
    rem_t = [min(max(rem - 256 * t, 0), 256) for t in range(8)]
    assert sum(rem_t) == rem and all(w % 128 == 0 for w in rem_t)
    assert tailw % 8 == 0 and (rem + tailw) % 8 == 0

    mesh = plsc.VectorSubcoreMesh(core_axis_name="c", subcore_axis_name="s")

    @functools.partial(
        pl.kernel,
        out_type=(jax.ShapeDtypeStruct((batch, vocab), jnp.float32),
                  jax.ShapeDtypeStruct((batch * tailw,), jnp.float32)),
        mesh=mesh,
        compiler_params=pltpu.CompilerParams(needs_layout_passes=False),
        scratch_types=[
            pltpu.VMEM((vocab + L,), jnp.float32),
            pltpu.VMEM((WIN,), jnp.int32),
            pltpu.VMEM((WIN,), jnp.float32),
            pltpu.VMEM((WIN,), jnp.float32),
            pltpu.VMEM((batch + L,), jnp.float32),
            pltpu.VMEM((batch + L,), jnp.int32),
            pltpu.VMEM_SHARED((2, 2, 8, WIN), jnp.int32),
            pltpu.VMEM_SHARED((2, 2, 8, WIN), jnp.float32),
            pltpu.SemaphoreType.DMA,
            pltpu.SemaphoreType.DMA,
        ],
    )
    def sc_kernel(sv_hbm, si_hbm, sit_hbm, p_hbm, k_hbm, out_hbm, ot_hbm,
                  row_v, idx_v, out_v0, out_v1, p_v, k_v,
                  shr_si, shr_out, sem_st, sem_fl):
        sid = lax.axis_index("s")
        cid = lax.axis_index("c")
        grp = sid // 8
        s8 = sid % 8
        pltpu.sync_copy(p_hbm, p_v.at[pl.ds(0, batch)])
        pltpu.sync_copy(k_hbm, k_v.at[pl.ds(0, batch)])

        out_v = (out_v0, out_v1)

        for r in range(rows_per_w):
            blk = cid * 4 + r * 2 + grp
            row = blk * 8 + s8
            base = row * vocab

            def stage_full(w, b):
                col = pl.multiple_of(w * WIN + s8 * SL, 128)
                pltpu.async_copy(
                    si_hbm.at[pl.ds(blk * 8, 8), pl.ds(col, SL)],
                    shr_si.at[b, grp, :, pl.ds(s8 * SL, SL)], sem_st)

            def wait_full():
                pltpu.make_async_copy(
                    si_hbm.at[pl.ds(blk * 8, 8), pl.ds(0, SL)],
                    shr_si.at[0, grp, :, pl.ds(0, SL)], sem_st).wait()

            def stage_rem():
                for tt, wt in enumerate(rem_t):
                    if wt == 0:
                        continue

                    @pl.when(s8 == tt)
                    def _():
                        pltpu.async_copy(
                            si_hbm.at[pl.ds(blk * 8, 8),
                                      pl.ds(nfull * WIN + 256 * tt, wt)],
                            shr_si.at[0, grp, :, pl.ds(256 * tt, wt)], sem_st)

            def wait_rem():
                for tt, wt in enumerate(rem_t):
                    if wt == 0:
                        continue

                    @pl.when(s8 == tt)
                    def _():
                        pltpu.make_async_copy(
                            si_hbm.at[pl.ds(blk * 8, 8), pl.ds(0, wt)],
                            shr_si.at[0, grp, :, pl.ds(0, wt)],
                            sem_st).wait()

            def fill(b, n):
                pltpu.sync_copy(out_v[b].at[pl.ds(0, n)],
                                shr_out.at[b, grp, s8, pl.ds(0, n)])

            def flush(w, b):
                col = pl.multiple_of(w * WIN + s8 * SL, 128)
                pltpu.async_copy(
                    shr_out.at[b, grp, :, pl.ds(s8 * SL, SL)],
                    out_hbm.at[pl.ds(blk * 8, 8), pl.ds(col, SL)], sem_fl)

            def wait_flush():
                pltpu.make_async_copy(
                    shr_out.at[0, grp, :, pl.ds(0, SL)],
                    out_hbm.at[pl.ds(blk * 8, 8), pl.ds(0, SL)],
                    sem_fl).wait()

            def flush_rem():
                for tt, wt in enumerate(rem_t):
                    if wt == 0:
                        continue

                    @pl.when(s8 == tt)
                    def _():
                        pltpu.async_copy(
                            shr_out.at[0, grp, :, pl.ds(256 * tt, wt)],
                            out_hbm.at[pl.ds(blk * 8, 8),
                                       pl.ds(nfull * WIN + 256 * tt, wt)],
                            sem_fl)

            def wait_flush_rem():
                for tt, wt in enumerate(rem_t):
                    if wt == 0:
                        continue

                    @pl.when(s8 == tt)
                    def _():
                        pltpu.make_async_copy(
                            shr_out.at[0, grp, :, pl.ds(0, wt)],
                            out_hbm.at[pl.ds(blk * 8, 8), pl.ds(0, wt)],
                            sem_fl).wait()

            stage_full(0, 0)
            stage_full(1, 1)
            pltpu.sync_copy(sv_hbm.at[pl.ds(base, vocab)],
                            row_v.at[pl.ds(0, vocab)])

            kk = _scalar_at(k_v, row)
            pp = _scalar_at(p_v, row)
            m = row_v[pl.ds(vocab - L, L)][L - 1]

            valid = kk >= 1
            idx_t = jnp.clip(vocab - kk, 0, vocab - 1)
            thresh = _scalar_at(row_v, idx_t)
            lo = jnp.int32(0)
            hi = jnp.int32(vocab)
            for _ in range(17):
                cont = lo < hi
                mid = (lo + hi) // 2
                vm = _scalar_at(row_v, jnp.minimum(mid, vocab - 1))
                below = vm < thresh
                lo = jnp.where(cont & below, mid + 1, lo)
                hi = jnp.where(cont & (~below), mid, hi)
            start = jnp.where(valid, lo, 0)
            g0 = start // L

            def sweep_a(g, acc):
                vv = row_v[pl.ds(g * L, L)]
                jj = lax.iota(jnp.int32, L) + g * L
                e = jnp.where(jj >= start, jnp.exp(vv - m), 0.0)
                return acc + e

            acc = lax.fori_loop(g0, nvreg, sweep_a,
                                jnp.zeros((L,), jnp.float32))
            total = jnp.sum(acc)
            t = (1.0 - pp) * total

            def sweep_b(g, carry):
                s, cnt = carry
                vv = row_v[pl.ds(g * L, L)]
                jj = lax.iota(jnp.int32, L) + g * L
                e = jnp.where(jj >= start, jnp.exp(vv - m), 0.0)
                pc = plsc.cumsum(e) + s
                cond = (pc <= t) & (jj >= start)
                cnt = cnt + jnp.sum(cond.astype(jnp.int32))
                return s + jnp.sum(e), cnt

            _, cnt = lax.fori_loop(g0, nvreg, sweep_b,
                                   (jnp.float32(0.0), jnp.int32(0)))
            c = start + cnt

            def extract(b, n):
                pltpu.sync_copy(shr_si.at[b, grp, s8, pl.ds(0, n)],
                                idx_v.at[pl.ds(0, n)])

            def gather(b, n):
                @plsc.parallel_loop(0, n, step=L, unroll=8)
                def gbody(i):
                    idx16 = idx_v[pl.ds(i, L)]
                    vals = plsc.load_gather(row_v, [idx16])
                    keep = (idx16 >= c) | (idx16 == vocab - 1)
                    out_v[b][pl.ds(i, L)] = jnp.where(keep, vals, NEG_INF)

            wait_full()
            plsc.subcore_barrier()

            def pair(q, _):
                w0 = 2 * q
                @pl.when(q > 0)
                def _():
                    stage_full(w0 + 1, 1)
                    flush(w0 - 1, 1)

                extract(0, WIN)
                gather(0, WIN)
                fill(0, WIN)
                wait_full()

                @pl.when(q > 0)
                def _():
                    wait_flush()

                plsc.subcore_barrier()

                stage_full(w0 + 2, 0)
                flush(w0, 0)
                extract(1, WIN)
                gather(1, WIN)
                fill(1, WIN)
                wait_full()
                wait_flush()
                plsc.subcore_barrier()
                return 0

            lax.fori_loop(0, (nfull - 2) // 2, pair, 0)

            stage_full(nfull - 1, 1)
            flush(nfull - 3, 1)
            extract(0, WIN)
            gather(0, WIN)
            fill(0, WIN)
            wait_full()
            wait_flush()
            plsc.subcore_barrier()

            stage_rem()
            flush(nfull - 2, 0)
            extract(1, WIN)
            gather(1, WIN)
            fill(1, WIN)
            wait_rem()
            wait_flush()
            plsc.subcore_barrier()

            flush(nfull - 1, 1)
            extract(0, rem)
            pltpu.sync_copy(sit_hbm.at[pl.ds(row * tailw, tailw)],
                            idx_v.at[pl.ds(rem, tailw)])
            gather(0, rem + tailw)
            fill(0, rem)
            pltpu.sync_copy(out_v[0].at[pl.ds(rem, tailw)],
                            ot_hbm.at[pl.ds(row * tailw, tailw)])
            wait_flush()
            plsc.subcore_barrier()

            flush_rem()
            wait_flush_rem()


    return sc_kernel


def kernel(sorted_value, sorted_indices, p, k):
    batch, vocab = sorted_value.shape
    fn = _build(batch, vocab)
    acols = (vocab // 128) * 128
    tailw = vocab - acols
    si = sorted_indices.astype(jnp.int32)
    out2d, otail = fn(sorted_value.reshape(-1), si, si[:, acols:].reshape(-1),
                      p.astype(jnp.float32), k.astype(jnp.int32))
    return lax.dynamic_update_slice(out2d, otail.reshape(batch, tailw),
                                    (0, acols))
